# Initial kernel scaffold; baseline (speedup 1.0000x reference)
import jax
import jax.numpy as jnp
from jax.experimental import pallas as pl


def kernel(x, Wg, bg, Wn, bn):
    raise NotImplementedError("write your pallas kernel here")


# speedup vs baseline: 10.0604x; 10.0604x over previous
import jax
import jax.numpy as jnp
from jax.experimental import pallas as pl


def _body(x_ref, wg_ref, bg_ref, wn_ref, bn_ref, o_ref):
    xb = x_ref[0]
    gate = jnp.dot(xb, wg_ref[...],
                   preferred_element_type=jnp.float32) + bg_ref[0, 0]
    m = jnp.max(gate)
    e = jnp.exp(gate - m)
    s = jnp.sum(e)
    pooled = jnp.dot(e.T, xb,
                     preferred_element_type=jnp.float32)
    inv = 1.0 / (s + 1e-16)
    out = jnp.dot(pooled * inv, wn_ref[...],
                  preferred_element_type=jnp.float32) + bn_ref[...] * (s * inv)
    o_ref[0] = out


def kernel(x, Wg, bg, Wn, bn):
    bz, n, f = x.shape
    bg2 = bg.reshape(1, 1)
    bn2 = bn.reshape(1, f)
    grid = (bz,)
    return pl.pallas_call(
        _body,
        grid=grid,
        in_specs=[
            pl.BlockSpec((1, n, f), lambda b: (b, 0, 0)),
            pl.BlockSpec((f, 1), lambda b: (0, 0)),
            pl.BlockSpec((1, 1), lambda b: (0, 0)),
            pl.BlockSpec((f, f), lambda b: (0, 0)),
            pl.BlockSpec((1, f), lambda b: (0, 0)),
        ],
        out_specs=pl.BlockSpec((1, 1, f), lambda b: (b, 0, 0)),
        out_shape=jax.ShapeDtypeStruct((bz, 1, f), jnp.float32),
    )(x, Wg, bg2, Wn, bn2).reshape(bz, f)


# speedup vs baseline: 12.5208x; 1.2446x over previous
import jax
import jax.numpy as jnp
from jax.experimental import pallas as pl


def _body(x_ref, wg_ref, bg_ref, wn_ref, bn_ref, o_ref):
    xb = x_ref[...]
    gate = jax.lax.dot_general(
        wg_ref[...], xb, (((1,), (1,)), ((), ())),
        preferred_element_type=jnp.float32) + bg_ref[0, 0]
    m = jnp.max(gate)
    e = jnp.exp(gate - m)
    s = jnp.sum(e)
    pooled = jnp.dot(e, xb,
                     preferred_element_type=jnp.float32)
    inv = 1.0 / (s + 1e-16)
    out = jnp.dot(pooled * inv, wn_ref[...],
                  preferred_element_type=jnp.float32) + bn_ref[...] * (s * inv)
    o_ref[0] = out


def kernel(x, Wg, bg, Wn, bn):
    bz, n, f = x.shape
    xf = x.reshape(bz * n, f)
    wgT = Wg.reshape(1, f)
    bg2 = bg.reshape(1, 1)
    bn2 = bn.reshape(1, f)
    grid = (bz,)
    return pl.pallas_call(
        _body,
        grid=grid,
        in_specs=[
            pl.BlockSpec((n, f), lambda b: (b, 0)),
            pl.BlockSpec((1, f), lambda b: (0, 0)),
            pl.BlockSpec((1, 1), lambda b: (0, 0)),
            pl.BlockSpec((f, f), lambda b: (0, 0)),
            pl.BlockSpec((1, f), lambda b: (0, 0)),
        ],
        out_specs=pl.BlockSpec((1, 1, f), lambda b: (b, 0, 0)),
        out_shape=jax.ShapeDtypeStruct((bz, 1, f), jnp.float32),
    )(xf, wgT, bg2, Wn, bn2).reshape(bz, f)
